# baseline (speedup 1.0000x reference)
import functools

import jax
import jax.numpy as jnp
from jax import lax
from jax.experimental import pallas as pl
from jax.experimental.pallas import tpu as pltpu
from jax.experimental.pallas import tpu_sc as plsc

NC = 2
NS = 16
NW = NC * NS
G = 128


@functools.partial(jax.jit, static_argnames=("chunk",))
def _sc_gather(table, idx_g, chunk):
    n_groups, _ = idx_g.shape
    B = n_groups * G
    D = table.shape[1]
    rows_per_w = B // NW
    groups_per_chunk = chunk // G
    n_chunks = rows_per_w // chunk
    assert rows_per_w % chunk == 0 and chunk % G == 0

    mesh = plsc.VectorSubcoreMesh(core_axis_name="c", subcore_axis_name="s")

    @functools.partial(
        pl.kernel,
        out_type=jax.ShapeDtypeStruct((B, D), jnp.float32),
        mesh=mesh,
        scratch_types=[
            pltpu.VMEM((groups_per_chunk, G), jnp.int32),
            pltpu.VMEM((chunk, D), jnp.float32),
            pltpu.SemaphoreType.DMA,
        ],
        compiler_params=pltpu.CompilerParams(use_tc_tiling_on_sc=False),
    )
    def k(table_hbm, idx_hbm, out_hbm, idx_v, rows_v, sem):
        wid = lax.axis_index("s") * NC + lax.axis_index("c")
        group_base = wid * (rows_per_w // G)
        row_base = wid * rows_per_w

        def body(c, _):
            pltpu.sync_copy(
                idx_hbm.at[pl.ds(group_base + c * groups_per_chunk,
                                 groups_per_chunk)],
                idx_v,
            )
            descs = [
                pltpu.async_copy(
                    table_hbm.at[idx_v.at[j]],
                    rows_v.at[pl.ds(j * G, G)],
                    sem,
                )
                for j in range(groups_per_chunk)
            ]
            for d in descs:
                d.wait()
            pltpu.sync_copy(rows_v, out_hbm.at[pl.ds(row_base + c * chunk, chunk)])
            return _

        lax.fori_loop(0, n_chunks, body, None)

    return k(table, idx_g)


def kernel(input_ids, embedding):
    B = input_ids.size
    idx_g = input_ids.astype(jnp.int32).reshape(B // G, G)
    out = _sc_gather(embedding, idx_g, chunk=1024)
    return out.reshape(*input_ids.shape, embedding.shape[1])


# speedup vs baseline: 1.0140x; 1.0140x over previous
import functools

import jax
import jax.numpy as jnp
from jax import lax
from jax.experimental import pallas as pl
from jax.experimental.pallas import tpu as pltpu
from jax.experimental.pallas import tpu_sc as plsc

NC = 2
NS = 16
NW = NC * NS
G = 128


@functools.partial(jax.jit, static_argnames=("chunk",))
def _sc_gather(table, idx_g, chunk):
    n_groups, _ = idx_g.shape
    B = n_groups * G
    D = table.shape[1]
    rows_per_w = B // NW
    groups_per_w = rows_per_w // G
    gpc = chunk // G
    n_chunks = rows_per_w // chunk
    n_pairs = n_chunks // 2
    assert rows_per_w % chunk == 0 and chunk % G == 0 and n_chunks % 2 == 0

    mesh = plsc.VectorSubcoreMesh(core_axis_name="c", subcore_axis_name="s")

    @functools.partial(
        pl.kernel,
        out_type=jax.ShapeDtypeStruct((B, D), jnp.float32),
        mesh=mesh,
        scratch_types=[
            pltpu.VMEM((groups_per_w, G), jnp.int32),
            pltpu.VMEM((chunk, D), jnp.float32),
            pltpu.VMEM((chunk, D), jnp.float32),
            pltpu.SemaphoreType.DMA,
            pltpu.SemaphoreType.DMA,
            pltpu.SemaphoreType.DMA,
            pltpu.SemaphoreType.DMA,
        ],
        compiler_params=pltpu.CompilerParams(use_tc_tiling_on_sc=False),
    )
    def k(table_hbm, idx_hbm, out_hbm, idx_v, rows0, rows1, g0, g1, s0, s1):
        wid = lax.axis_index("s") * NC + lax.axis_index("c")
        row_base = wid * rows_per_w

        pltpu.sync_copy(idx_hbm.at[pl.ds(wid * groups_per_w, groups_per_w)],
                        idx_v)

        rows = (rows0, rows1)
        gsem = (g0, g1)
        ssem = (s0, s1)

        def fire_gather(c, b):
            for j in range(gpc):
                pltpu.async_copy(table_hbm.at[idx_v.at[c * gpc + j]],
                                 rows[b].at[pl.ds(j * G, G)], gsem[b])

        def wait_gather(b):
            pltpu.make_async_copy(table_hbm.at[pl.ds(0, chunk)], rows[b],
                                  gsem[b]).wait()

        def fire_store(c, b):
            pltpu.async_copy(rows[b], out_hbm.at[pl.ds(row_base + c * chunk,
                                                       chunk)], ssem[b])

        def wait_store(b):
            pltpu.make_async_copy(rows[b], out_hbm.at[pl.ds(row_base, chunk)],
                                  ssem[b]).wait()

        fire_gather(0, 0)

        def body(i, carry):
            c0 = 2 * i
            wait_gather(0)
            fire_store(c0, 0)

            @pl.when(i > 0)
            def _():
                wait_store(1)

            fire_gather(c0 + 1, 1)
            wait_gather(1)
            fire_store(c0 + 1, 1)
            wait_store(0)

            @pl.when(i < n_pairs - 1)
            def _():
                fire_gather(c0 + 2, 0)

            return carry

        lax.fori_loop(0, n_pairs, body, None)
        wait_store(1)

    return k(table, idx_g)


def kernel(input_ids, embedding):
    B = input_ids.size
    idx_g = input_ids.astype(jnp.int32).reshape(B // G, G)
    out = _sc_gather(embedding, idx_g, chunk=512)
    return out.reshape(*input_ids.shape, embedding.shape[1])
